# Initial kernel scaffold; baseline (speedup 1.0000x reference)
import jax
import jax.numpy as jnp
from jax.experimental import pallas as pl


def kernel(base_indices, reservoir_encoded, embedding):
    raise NotImplementedError("write your pallas kernel here")


# speedup vs baseline: 29.1882x; 29.1882x over previous
import functools

import jax
import jax.numpy as jnp
from jax import lax
from jax.experimental import pallas as pl
from jax.experimental.pallas import tpu as pltpu
from jax.experimental.pallas import tpu_sc as plsc

_FROZEN = 0
_LANES = 16

_info = plsc.get_sparse_core_info()
_NC, _NS = _info.num_cores, _info.num_subcores
_NW = _NC * _NS


def _build_table_kernel(n_pad, res_len, feat):
    tpw = n_pad // _NW
    ids_pw = tpw * res_len
    idx_rows = ids_pw // 128
    mesh = plsc.VectorSubcoreMesh(core_axis_name="c", subcore_axis_name="s")

    @functools.partial(
        pl.kernel,
        mesh=mesh,
        out_type=jax.ShapeDtypeStruct((n_pad, feat), jnp.float32),
        compiler_params=pltpu.CompilerParams(use_tc_tiling_on_sc=False),
        scratch_types=[
            pltpu.VMEM((ids_pw + _LANES,), jnp.int32),
            pltpu.VMEM((ids_pw, feat), jnp.float32),
            pltpu.VMEM((tpw, feat), jnp.float32),
            pltpu.SemaphoreType.DMA,
        ],
    )
    def build(res_flat_hbm, emb_hbm, t_hbm, ids_v, rows_v, tloc_v, sem):
        wid = lax.axis_index("s") * _NC + lax.axis_index("c")
        pltpu.sync_copy(res_flat_hbm.at[pl.ds(wid * ids_pw, ids_pw)], ids_v.at[pl.ds(0, ids_pw)])
        for i in range(idx_rows):
            pltpu.async_copy(
                emb_hbm.at[ids_v.at[pl.ds(i * 128, 128)]],
                rows_v.at[pl.ds(i * 128, 128)],
                sem,
            ).wait()

        def token_body(j, carry):
            accs = [jnp.zeros((_LANES,), jnp.float32)] * (feat // _LANES)
            idchunk = ids_v[pl.ds(j * res_len, _LANES)]
            for r in range(res_len):
                row = j * res_len + r
                m = jnp.full(
                    (_LANES,),
                    jnp.where(idchunk[r] != _FROZEN, jnp.float32(1.0), jnp.float32(0.0)),
                )
                for c in range(feat // _LANES):
                    accs[c] = accs[c] + rows_v[row, pl.ds(c * _LANES, _LANES)] * m
            for c in range(feat // _LANES):
                tloc_v[j, pl.ds(c * _LANES, _LANES)] = accs[c]
            return carry

        lax.fori_loop(0, tpw, token_body, 0)
        pltpu.sync_copy(tloc_v, t_hbm.at[pl.ds(wid * tpw, tpw)])

    return build


def _gather_rows_kernel(n_rows, feat):
    rpw = n_rows // _NW
    chunks = rpw // 128
    mesh = plsc.VectorSubcoreMesh(core_axis_name="c", subcore_axis_name="s")

    @functools.partial(
        pl.kernel,
        mesh=mesh,
        out_type=jax.ShapeDtypeStruct((n_rows, feat), jnp.float32),
        compiler_params=pltpu.CompilerParams(use_tc_tiling_on_sc=False),
        scratch_types=[
            pltpu.VMEM((chunks, 128), jnp.int32),
            pltpu.VMEM((128, feat), jnp.float32),
            pltpu.SemaphoreType.DMA,
        ],
    )
    def gather(t_hbm, idx_hbm, out_hbm, idx_v, rows_v, sem):
        wid = lax.axis_index("s") * _NC + lax.axis_index("c")
        base = wid * rpw
        pltpu.sync_copy(idx_hbm.at[pl.ds(wid * chunks, chunks)], idx_v)

        def chunk_body(k, carry):
            pltpu.async_copy(t_hbm.at[idx_v.at[k]], rows_v, sem).wait()
            pltpu.sync_copy(rows_v, out_hbm.at[pl.ds(base + k * 128, 128)])
            return carry

        lax.fori_loop(0, chunks, chunk_body, 0)

    return gather


def kernel(base_indices, reservoir_encoded, embedding):
    batch, seq = base_indices.shape
    n_tokens, res_len = reservoir_encoded.shape
    vocab, feat = embedding.shape
    n_rows = batch * seq

    n_pad = ((n_tokens + _NW - 1) // _NW) * _NW
    if (n_pad * res_len) % (128 * _NW):
        n_pad = ((n_tokens * res_len + 128 * _NW - 1) // (128 * _NW)) * 128 * _NW // res_len

    res_pad = jnp.zeros((n_pad, res_len), jnp.int32).at[:n_tokens].set(reservoir_encoded)
    res_flat = res_pad.reshape(-1)

    table = _build_table_kernel(n_pad, res_len, feat)(res_flat, embedding)

    idx_2d = base_indices.reshape(-1, 128)
    out_flat = _gather_rows_kernel(n_rows, feat)(table, idx_2d)
    return out_flat.reshape(batch, seq, feat)


# speedup vs baseline: 30.6457x; 1.0499x over previous
import functools

import jax
import jax.numpy as jnp
from jax import lax
from jax.experimental import pallas as pl
from jax.experimental.pallas import tpu as pltpu
from jax.experimental.pallas import tpu_sc as plsc

_FROZEN = 0
_LANES = 16

_NBUF = 4

_info = plsc.get_sparse_core_info()
_NC, _NS = _info.num_cores, _info.num_subcores
_NW = _NC * _NS


def _build_table_kernel(n_pad, res_len, feat):
    tpw = n_pad // _NW
    ids_pw = tpw * res_len
    idx_rows = ids_pw // 128
    mesh = plsc.VectorSubcoreMesh(core_axis_name="c", subcore_axis_name="s")

    @functools.partial(
        pl.kernel,
        mesh=mesh,
        out_type=jax.ShapeDtypeStruct((n_pad, feat), jnp.float32),
        compiler_params=pltpu.CompilerParams(use_tc_tiling_on_sc=False),
        scratch_types=[
            pltpu.VMEM((ids_pw + _LANES,), jnp.int32),
            pltpu.VMEM((ids_pw, feat), jnp.float32),
            pltpu.VMEM((tpw, feat), jnp.float32),
            pltpu.SemaphoreType.DMA,
        ],
    )
    def build(res_flat_hbm, emb_hbm, t_hbm, ids_v, rows_v, tloc_v, sem):
        wid = lax.axis_index("s") * _NC + lax.axis_index("c")
        pltpu.sync_copy(res_flat_hbm.at[pl.ds(wid * ids_pw, ids_pw)], ids_v.at[pl.ds(0, ids_pw)])
        for i in range(idx_rows):
            pltpu.async_copy(
                emb_hbm.at[ids_v.at[pl.ds(i * 128, 128)]],
                rows_v.at[pl.ds(i * 128, 128)],
                sem,
            ).wait()

        def token_body(j, carry):
            accs = [jnp.zeros((_LANES,), jnp.float32)] * (feat // _LANES)
            idchunk = ids_v[pl.ds(j * res_len, _LANES)]
            for r in range(res_len):
                row = j * res_len + r
                m = jnp.full(
                    (_LANES,),
                    jnp.where(idchunk[r] != _FROZEN, jnp.float32(1.0), jnp.float32(0.0)),
                )
                for c in range(feat // _LANES):
                    accs[c] = accs[c] + rows_v[row, pl.ds(c * _LANES, _LANES)] * m
            for c in range(feat // _LANES):
                tloc_v[j, pl.ds(c * _LANES, _LANES)] = accs[c]
            return carry

        lax.fori_loop(0, tpw, token_body, 0)
        pltpu.sync_copy(tloc_v, t_hbm.at[pl.ds(wid * tpw, tpw)])

    return build


def _gather_rows_kernel(n_rows, feat):
    rpw = n_rows // _NW
    chunks = rpw // 128
    mesh = plsc.VectorSubcoreMesh(core_axis_name="c", subcore_axis_name="s")

    @functools.partial(
        pl.kernel,
        mesh=mesh,
        out_type=jax.ShapeDtypeStruct((n_rows, feat), jnp.float32),
        compiler_params=pltpu.CompilerParams(use_tc_tiling_on_sc=False),
        scratch_types=[
            pltpu.VMEM((chunks, 128), jnp.int32),
            pltpu.VMEM((_NBUF, 128, feat), jnp.float32),
            [pltpu.SemaphoreType.DMA] * _NBUF,
            [pltpu.SemaphoreType.DMA] * _NBUF,
        ],
    )
    def gather(t_hbm, idx_hbm, out_hbm, idx_v, rows_v, gsems, wsems):
        wid = lax.axis_index("s") * _NC + lax.axis_index("c")
        base = wid * rpw
        pltpu.sync_copy(idx_hbm.at[pl.ds(wid * chunks, chunks)], idx_v)

        def start_gather(k):
            p = k % _NBUF
            return pltpu.async_copy(t_hbm.at[idx_v.at[k]], rows_v.at[p], gsems[p])

        def start_write(k):
            p = k % _NBUF
            return pltpu.async_copy(
                rows_v.at[p], out_hbm.at[pl.ds(base + k * 128, 128)], wsems[p]
            )

        ghandles = {0: start_gather(0)}
        whandles = {}
        for k in range(chunks):
            if k + 1 < chunks:
                if k + 1 >= _NBUF:
                    whandles.pop(k + 1 - _NBUF).wait()
                ghandles[k + 1] = start_gather(k + 1)
            ghandles.pop(k).wait()
            whandles[k] = start_write(k)
        for k in sorted(whandles):
            whandles.pop(k).wait()

    return gather


def kernel(base_indices, reservoir_encoded, embedding):
    batch, seq = base_indices.shape
    n_tokens, res_len = reservoir_encoded.shape
    vocab, feat = embedding.shape
    n_rows = batch * seq

    n_pad = ((n_tokens + _NW - 1) // _NW) * _NW
    if (n_pad * res_len) % (128 * _NW):
        n_pad = ((n_tokens * res_len + 128 * _NW - 1) // (128 * _NW)) * 128 * _NW // res_len

    res_pad = jnp.zeros((n_pad, res_len), jnp.int32).at[:n_tokens].set(reservoir_encoded)
    res_flat = res_pad.reshape(-1)

    table = _build_table_kernel(n_pad, res_len, feat)(res_flat, embedding)

    idx_2d = base_indices.reshape(-1, 128)
    out_flat = _gather_rows_kernel(n_rows, feat)(table, idx_2d)
    return out_flat.reshape(batch, seq, feat)
